# Initial kernel scaffold; baseline (speedup 1.0000x reference)
import jax
import jax.numpy as jnp
from jax.experimental import pallas as pl


def kernel(target, input, weight, bias, noise, noise_idx):
    raise NotImplementedError("write your pallas kernel here")


# speedup vs baseline: 1.5922x; 1.5922x over previous
import functools
import math

import jax
import jax.numpy as jnp
from jax import lax
from jax.experimental import pallas as pl
from jax.experimental.pallas import tpu as pltpu
from jax.experimental.pallas import tpu_sc as plsc

_V = 100000
_D = 1024
_K = 100
_KPAD = 128
_BN = 1024
_NORM = math.log(_V)
_EPS = 1e-10
_NC = 2
_NS = 16
_NW = _NC * _NS
_TPW = _BN // _NW
_L = 16
_KROWS = _KPAD // _NS


def _sc_body(weight_hbm, bias_hbm, noise_hbm, target_hbm, x_hbm, nidx_hbm,
             tscore_hbm, ptn_hbm, wn_hbm, bn_hbm, nn_hbm,
             idx_v, rows_v, x_v, bias_v, ptn_v, partial_v, score_v,
             kidx_v, wnrows_v, bn_v, nn_v, sem_rows, sem_x):
    cid = lax.axis_index("c")
    sid = lax.axis_index("s")
    wid = sid * _NC + cid
    base = wid * _TPW

    pltpu.sync_copy(target_hbm.at[pl.ds(base, _TPW)], idx_v)
    rows_cp = pltpu.async_copy(weight_hbm.at[idx_v], rows_v, sem_rows)
    x_cp = pltpu.async_copy(x_hbm.at[pl.ds(base, _TPW), :], x_v, sem_x)
    pltpu.sync_copy(bias_hbm.at[idx_v], bias_v)
    pltpu.sync_copy(noise_hbm.at[idx_v], ptn_v)
    pltpu.sync_copy(ptn_v, ptn_hbm.at[pl.ds(base, _TPW)])

    @pl.when(wid < _NS)
    def _():
        kbase = wid * _KROWS
        pltpu.sync_copy(nidx_hbm.at[pl.ds(kbase, _KROWS)], kidx_v)
        pltpu.sync_copy(weight_hbm.at[kidx_v], wnrows_v)
        pltpu.sync_copy(wnrows_v, wn_hbm.at[pl.ds(kbase, _KROWS), :])
        pltpu.sync_copy(bias_hbm.at[kidx_v], bn_v)
        pltpu.sync_copy(bn_v, bn_hbm.at[pl.ds(kbase, _KROWS)])
        pltpu.sync_copy(noise_hbm.at[kidx_v], nn_v)
        pltpu.sync_copy(nn_v, nn_hbm.at[pl.ds(kbase, _KROWS)])

    rows_cp.wait()
    x_cp.wait()

    def tok_body(t, carry):
        acc = jnp.zeros((_L,), jnp.float32)
        for j in range(_D // _L):
            acc = acc + rows_v[t, pl.ds(j * _L, _L)] * x_v[t, pl.ds(j * _L, _L)]
        partial_v[t, :] = acc
        return carry

    lax.fori_loop(0, _TPW, tok_body, 0)

    lanes = lax.iota(jnp.int32, _L)
    for g in range(_TPW // _L):
        row_ids = lanes + g * _L
        acc16 = jnp.zeros((_L,), jnp.float32)
        for j in range(_L):
            col_ids = jnp.full((_L,), j, jnp.int32)
            acc16 = acc16 + plsc.load_gather(partial_v, [row_ids, col_ids])
        score_v[pl.ds(g * _L, _L)] = acc16 + bias_v[pl.ds(g * _L, _L)]

    pltpu.sync_copy(score_v, tscore_hbm.at[pl.ds(base, _TPW)])


_sc_gather = functools.partial(
    pl.kernel,
    out_type=[
        jax.ShapeDtypeStruct((_BN,), jnp.float32),
        jax.ShapeDtypeStruct((_BN,), jnp.float32),
        jax.ShapeDtypeStruct((_KPAD, _D), jnp.float32),
        jax.ShapeDtypeStruct((_KPAD,), jnp.float32),
        jax.ShapeDtypeStruct((_KPAD,), jnp.float32),
    ],
    mesh=plsc.VectorSubcoreMesh(core_axis_name="c", subcore_axis_name="s"),
    compiler_params=pltpu.CompilerParams(needs_layout_passes=False),
    scratch_types=[
        pltpu.VMEM((_TPW,), jnp.int32),
        pltpu.VMEM((_TPW, _D), jnp.float32),
        pltpu.VMEM((_TPW, _D), jnp.float32),
        pltpu.VMEM((_TPW,), jnp.float32),
        pltpu.VMEM((_TPW,), jnp.float32),
        pltpu.VMEM((_TPW, _L), jnp.float32),
        pltpu.VMEM((_TPW,), jnp.float32),
        pltpu.VMEM((_KROWS,), jnp.int32),
        pltpu.VMEM((_KROWS, _D), jnp.float32),
        pltpu.VMEM((_KROWS,), jnp.float32),
        pltpu.VMEM((_KROWS,), jnp.float32),
        pltpu.SemaphoreType.DMA,
        pltpu.SemaphoreType.DMA,
    ],
)(_sc_body)


def _tc_body(x_ref, wn_ref, bn_ref, nn_ref, ts_ref, ptn_ref, out_ref):
    xs = x_ref[...]
    wns = wn_ref[...]
    ns = lax.dot_general(xs, wns, (((1,), (1,)), ((), ())),
                         preferred_element_type=jnp.float32)
    ns = ns + bn_ref[...]
    pnim = jnp.exp(jnp.minimum(ns - _NORM, 20.0))
    kpn = _K * nn_ref[...]
    pnc = kpn / (pnim + kpn + _EPS)
    lg = jnp.log(pnc + _EPS)
    kmask = lax.broadcasted_iota(jnp.int32, (_BN, _KPAD), 1) < _K
    term2 = jnp.sum(jnp.where(kmask, lg, 0.0))
    pm = jnp.exp(jnp.minimum(ts_ref[...] - _NORM, 20.0))
    pt = pm / (pm + _K * ptn_ref[...] + _EPS)
    term1 = jnp.sum(jnp.log(pt + _EPS))
    out_ref[...] = jnp.reshape(-(term1 + term2) / _BN, (1, 1))


def _tc_loss(x2d, wn, bn2, nn2, ts2, ptn2):
    return pl.pallas_call(
        _tc_body,
        out_shape=jax.ShapeDtypeStruct((1, 1), jnp.float32),
    )(x2d, wn, bn2, nn2, ts2, ptn2)


def kernel(target, input, weight, bias, noise, noise_idx):
    x2d = input.reshape(_BN, _D)
    tflat = target.reshape(_BN).astype(jnp.int32)
    nidx = jnp.concatenate([noise_idx.astype(jnp.int32),
                            jnp.zeros((_KPAD - _K,), jnp.int32)])
    tscore, ptn, wn, bn, nn = _sc_gather(weight, bias, noise, tflat, x2d, nidx)
    out = _tc_loss(x2d, wn, bn.reshape(1, _KPAD), nn.reshape(1, _KPAD),
                   tscore.reshape(8, 128), ptn.reshape(8, 128))
    return out[0, 0]


# speedup vs baseline: 1.6225x; 1.0190x over previous
import functools
import math

import jax
import jax.numpy as jnp
from jax import lax
from jax.experimental import pallas as pl
from jax.experimental.pallas import tpu as pltpu
from jax.experimental.pallas import tpu_sc as plsc

_V = 100000
_D = 1024
_K = 100
_KPAD = 128
_BN = 1024
_NORM = math.log(_V)
_EPS = 1e-10
_NC = 2
_NS = 16
_NW = _NC * _NS
_TPW = _BN // _NW
_L = 16
_KROWS = _KPAD // _NS
_NCH = 4
_CT = _TPW // _NCH


def _sc_body(weight_hbm, bias_hbm, noise_hbm, target_hbm, x_hbm, nidx_hbm,
             tscore_hbm, ptn_hbm, wn_hbm, bn_hbm, nn_hbm,
             idx_v, rows_v, x_v, bias_v, ptn_v, partial_v, score_v,
             kidx_v, wnrows_v, bn_v, nn_v,
             sem_idx, sem_kidx, sem_rows, sem_x, sem_bias, sem_ptn,
             sem_wn, sem_bn, sem_nn, sem_po, sem_wno, sem_bno, sem_nno):
    cid = lax.axis_index("c")
    sid = lax.axis_index("s")
    wid = sid * _NC + cid
    base = wid * _TPW
    kbase = wid * _KROWS
    noise_worker = wid < _NS

    idx_cp = pltpu.async_copy(target_hbm.at[pl.ds(base, _TPW)], idx_v, sem_idx)

    @pl.when(noise_worker)
    def _():
        pltpu.async_copy(nidx_hbm.at[pl.ds(kbase, _KROWS)], kidx_v, sem_kidx)

    idx_cp.wait()

    rows_cps = []
    x_cps = []
    for c in range(_NCH):
        rows_cps.append(pltpu.async_copy(
            weight_hbm.at[idx_v.at[pl.ds(c * _CT, _CT)]],
            rows_v.at[pl.ds(c * _CT, _CT), :], sem_rows[c]))
        x_cps.append(pltpu.async_copy(
            x_hbm.at[pl.ds(base + c * _CT, _CT), :],
            x_v.at[pl.ds(c * _CT, _CT), :], sem_x[c]))
    bias_cp = pltpu.async_copy(bias_hbm.at[idx_v], bias_v, sem_bias)
    ptn_cp = pltpu.async_copy(noise_hbm.at[idx_v], ptn_v, sem_ptn)

    @pl.when(noise_worker)
    def _():
        pltpu.make_async_copy(nidx_hbm.at[pl.ds(kbase, _KROWS)],
                              kidx_v, sem_kidx).wait()
        pltpu.async_copy(weight_hbm.at[kidx_v], wnrows_v, sem_wn)
        pltpu.async_copy(bias_hbm.at[kidx_v], bn_v, sem_bn)
        pltpu.async_copy(noise_hbm.at[kidx_v], nn_v, sem_nn)

    for c in range(_NCH):
        rows_cps[c].wait()
        x_cps[c].wait()

        def tok_body(t, carry):
            acc = jnp.zeros((_L,), jnp.float32)
            for j in range(_D // _L):
                acc = acc + (rows_v[t, pl.ds(j * _L, _L)] *
                             x_v[t, pl.ds(j * _L, _L)])
            partial_v[t, :] = acc
            return carry

        lax.fori_loop(c * _CT, (c + 1) * _CT, tok_body, 0)

    @pl.when(noise_worker)
    def _():
        pltpu.make_async_copy(weight_hbm.at[kidx_v], wnrows_v, sem_wn).wait()
        pltpu.make_async_copy(bias_hbm.at[kidx_v], bn_v, sem_bn).wait()
        pltpu.make_async_copy(noise_hbm.at[kidx_v], nn_v, sem_nn).wait()
        pltpu.async_copy(wnrows_v, wn_hbm.at[pl.ds(kbase, _KROWS), :], sem_wno)
        pltpu.async_copy(bn_v, bn_hbm.at[pl.ds(kbase, _KROWS)], sem_bno)
        pltpu.async_copy(nn_v, nn_hbm.at[pl.ds(kbase, _KROWS)], sem_nno)

    ptn_cp.wait()
    pltpu.async_copy(ptn_v, ptn_hbm.at[pl.ds(base, _TPW)], sem_po)
    bias_cp.wait()

    lanes = lax.iota(jnp.int32, _L)
    for g in range(_TPW // _L):
        row_ids = lanes + g * _L
        acc16 = jnp.zeros((_L,), jnp.float32)
        for j in range(_L):
            col_ids = jnp.full((_L,), j, jnp.int32)
            acc16 = acc16 + plsc.load_gather(partial_v, [row_ids, col_ids])
        score_v[pl.ds(g * _L, _L)] = acc16 + bias_v[pl.ds(g * _L, _L)]

    pltpu.sync_copy(score_v, tscore_hbm.at[pl.ds(base, _TPW)])
    pltpu.make_async_copy(ptn_v, ptn_hbm.at[pl.ds(base, _TPW)], sem_po).wait()

    @pl.when(noise_worker)
    def _():
        pltpu.make_async_copy(
            wnrows_v, wn_hbm.at[pl.ds(kbase, _KROWS), :], sem_wno).wait()
        pltpu.make_async_copy(
            bn_v, bn_hbm.at[pl.ds(kbase, _KROWS)], sem_bno).wait()
        pltpu.make_async_copy(
            nn_v, nn_hbm.at[pl.ds(kbase, _KROWS)], sem_nno).wait()


_sc_gather = functools.partial(
    pl.kernel,
    out_type=[
        jax.ShapeDtypeStruct((_BN,), jnp.float32),
        jax.ShapeDtypeStruct((_BN,), jnp.float32),
        jax.ShapeDtypeStruct((_KPAD, _D), jnp.float32),
        jax.ShapeDtypeStruct((_KPAD,), jnp.float32),
        jax.ShapeDtypeStruct((_KPAD,), jnp.float32),
    ],
    mesh=plsc.VectorSubcoreMesh(core_axis_name="c", subcore_axis_name="s"),
    compiler_params=pltpu.CompilerParams(needs_layout_passes=False),
    scratch_types=[
        pltpu.VMEM((_TPW,), jnp.int32),
        pltpu.VMEM((_TPW, _D), jnp.float32),
        pltpu.VMEM((_TPW, _D), jnp.float32),
        pltpu.VMEM((_TPW,), jnp.float32),
        pltpu.VMEM((_TPW,), jnp.float32),
        pltpu.VMEM((_TPW, _L), jnp.float32),
        pltpu.VMEM((_TPW,), jnp.float32),
        pltpu.VMEM((_KROWS,), jnp.int32),
        pltpu.VMEM((_KROWS, _D), jnp.float32),
        pltpu.VMEM((_KROWS,), jnp.float32),
        pltpu.VMEM((_KROWS,), jnp.float32),
        pltpu.SemaphoreType.DMA,
        pltpu.SemaphoreType.DMA,
        [pltpu.SemaphoreType.DMA] * _NCH,
        [pltpu.SemaphoreType.DMA] * _NCH,
        pltpu.SemaphoreType.DMA,
        pltpu.SemaphoreType.DMA,
        pltpu.SemaphoreType.DMA,
        pltpu.SemaphoreType.DMA,
        pltpu.SemaphoreType.DMA,
        pltpu.SemaphoreType.DMA,
        pltpu.SemaphoreType.DMA,
        pltpu.SemaphoreType.DMA,
        pltpu.SemaphoreType.DMA,
    ],
)(_sc_body)


def _tc_body(x_ref, wn_ref, bn_ref, nn_ref, ts_ref, ptn_ref, out_ref):
    xs = x_ref[...]
    wns = wn_ref[...]
    ns = lax.dot_general(xs, wns, (((1,), (1,)), ((), ())),
                         preferred_element_type=jnp.float32)
    ns = ns + bn_ref[...]
    pnim = jnp.exp(jnp.minimum(ns - _NORM, 20.0))
    kpn = _K * nn_ref[...]
    pnc = kpn / (pnim + kpn + _EPS)
    lg = jnp.log(pnc + _EPS)
    kmask = lax.broadcasted_iota(jnp.int32, (_BN, _KPAD), 1) < _K
    term2 = jnp.sum(jnp.where(kmask, lg, 0.0))
    pm = jnp.exp(jnp.minimum(ts_ref[...] - _NORM, 20.0))
    pt = pm / (pm + _K * ptn_ref[...] + _EPS)
    term1 = jnp.sum(jnp.log(pt + _EPS))
    out_ref[...] = jnp.reshape(-(term1 + term2) / _BN, (1, 1))


def _tc_loss(x2d, wn, bn2, nn2, ts2, ptn2):
    return pl.pallas_call(
        _tc_body,
        out_shape=jax.ShapeDtypeStruct((1, 1), jnp.float32),
    )(x2d, wn, bn2, nn2, ts2, ptn2)


def kernel(target, input, weight, bias, noise, noise_idx):
    x2d = input.reshape(_BN, _D)
    tflat = target.reshape(_BN).astype(jnp.int32)
    nidx = jnp.concatenate([noise_idx.astype(jnp.int32),
                            jnp.zeros((_KPAD - _K,), jnp.int32)])
    tscore, ptn, wn, bn, nn = _sc_gather(weight, bias, noise, tflat, x2d, nidx)
    out = _tc_loss(x2d, wn, bn.reshape(1, _KPAD), nn.reshape(1, _KPAD),
                   tscore.reshape(8, 128), ptn.reshape(8, 128))
    return out[0, 0]


# speedup vs baseline: 1.8577x; 1.1449x over previous
import functools
import math

import jax
import jax.numpy as jnp
from jax import lax
from jax.experimental import pallas as pl
from jax.experimental.pallas import tpu as pltpu
from jax.experimental.pallas import tpu_sc as plsc

_V = 100000
_D = 1024
_K = 100
_KPAD = 128
_BN = 1024
_NORM = math.log(_V)
_EPS = 1e-10
_NC = 2
_NS = 16
_NW = _NC * _NS
_TPW = _BN // _NW
_L = 16
_NCH = 4
_CT = _TPW // _NCH
_JU = 8


def _sc_body(weight_hbm, bias_hbm, noise_hbm, target_hbm, x_hbm, nidx_hbm,
             tscore_hbm, ptn_hbm, bn_hbm, nn_hbm,
             idx_v, rows_v, x_v, bias_v, ptn_v, partial_v, score_v,
             kidx_v, bn_v, nn_v,
             sem_idx, sem_kidx, sem_rows, sem_x, sem_bias, sem_ptn,
             sem_bn, sem_nn, sem_po, sem_bno, sem_nno):
    cid = lax.axis_index("c")
    sid = lax.axis_index("s")
    wid = sid * _NC + cid
    base = wid * _TPW
    noise_worker = wid == _NW - 1

    idx_cp = pltpu.async_copy(target_hbm.at[pl.ds(base, _TPW)], idx_v, sem_idx)

    @pl.when(noise_worker)
    def _():
        pltpu.async_copy(nidx_hbm, kidx_v, sem_kidx)

    idx_cp.wait()

    rows_cps = []
    x_cps = []
    for c in range(_NCH):
        rows_cps.append(pltpu.async_copy(
            weight_hbm.at[idx_v.at[pl.ds(c * _CT, _CT)]],
            rows_v.at[pl.ds(c * _CT, _CT), :], sem_rows[c]))
        x_cps.append(pltpu.async_copy(
            x_hbm.at[pl.ds(base + c * _CT, _CT), :],
            x_v.at[pl.ds(c * _CT, _CT), :], sem_x[c]))
    bias_cp = pltpu.async_copy(bias_hbm.at[idx_v], bias_v, sem_bias)
    ptn_cp = pltpu.async_copy(noise_hbm.at[idx_v], ptn_v, sem_ptn)

    @pl.when(noise_worker)
    def _():
        pltpu.make_async_copy(nidx_hbm, kidx_v, sem_kidx).wait()
        pltpu.async_copy(bias_hbm.at[kidx_v], bn_v, sem_bn)
        pltpu.async_copy(noise_hbm.at[kidx_v], nn_v, sem_nn)

    for c in range(_NCH):
        rows_cps[c].wait()
        x_cps[c].wait()

        def tok_body(t, carry):
            def j_body(jo, acc):
                for u in range(_JU):
                    off = jo * (_JU * _L) + u * _L
                    acc = acc + (rows_v[t, pl.ds(off, _L)] *
                                 x_v[t, pl.ds(off, _L)])
                return acc

            acc = lax.fori_loop(0, _D // (_JU * _L), j_body,
                                jnp.zeros((_L,), jnp.float32))
            partial_v[t, :] = acc
            return carry

        lax.fori_loop(c * _CT, (c + 1) * _CT, tok_body, 0)

    @pl.when(noise_worker)
    def _():
        pltpu.make_async_copy(bias_hbm.at[kidx_v], bn_v, sem_bn).wait()
        pltpu.make_async_copy(noise_hbm.at[kidx_v], nn_v, sem_nn).wait()
        pltpu.async_copy(bn_v, bn_hbm, sem_bno)
        pltpu.async_copy(nn_v, nn_hbm, sem_nno)

    ptn_cp.wait()
    pltpu.async_copy(ptn_v, ptn_hbm.at[pl.ds(base, _TPW)], sem_po)
    bias_cp.wait()

    lanes = lax.iota(jnp.int32, _L)
    for g in range(_TPW // _L):
        row_ids = lanes + g * _L
        acc16 = jnp.zeros((_L,), jnp.float32)
        for j in range(_L):
            col_ids = jnp.full((_L,), j, jnp.int32)
            acc16 = acc16 + plsc.load_gather(partial_v, [row_ids, col_ids])
        score_v[pl.ds(g * _L, _L)] = acc16 + bias_v[pl.ds(g * _L, _L)]

    pltpu.sync_copy(score_v, tscore_hbm.at[pl.ds(base, _TPW)])
    pltpu.make_async_copy(ptn_v, ptn_hbm.at[pl.ds(base, _TPW)], sem_po).wait()

    @pl.when(noise_worker)
    def _():
        pltpu.make_async_copy(bn_v, bn_hbm, sem_bno).wait()
        pltpu.make_async_copy(nn_v, nn_hbm, sem_nno).wait()


_sc_gather = functools.partial(
    pl.kernel,
    out_type=[
        jax.ShapeDtypeStruct((_BN,), jnp.float32),
        jax.ShapeDtypeStruct((_BN,), jnp.float32),
        jax.ShapeDtypeStruct((_K,), jnp.float32),
        jax.ShapeDtypeStruct((_K,), jnp.float32),
    ],
    mesh=plsc.VectorSubcoreMesh(core_axis_name="c", subcore_axis_name="s"),
    compiler_params=pltpu.CompilerParams(needs_layout_passes=False),
    scratch_types=[
        pltpu.VMEM((_TPW,), jnp.int32),
        pltpu.VMEM((_TPW, _D), jnp.float32),
        pltpu.VMEM((_TPW, _D), jnp.float32),
        pltpu.VMEM((_TPW,), jnp.float32),
        pltpu.VMEM((_TPW,), jnp.float32),
        pltpu.VMEM((_TPW, _L), jnp.float32),
        pltpu.VMEM((_TPW,), jnp.float32),
        pltpu.VMEM((_K,), jnp.int32),
        pltpu.VMEM((_K,), jnp.float32),
        pltpu.VMEM((_K,), jnp.float32),
        pltpu.SemaphoreType.DMA,
        pltpu.SemaphoreType.DMA,
        [pltpu.SemaphoreType.DMA] * _NCH,
        [pltpu.SemaphoreType.DMA] * _NCH,
        pltpu.SemaphoreType.DMA,
        pltpu.SemaphoreType.DMA,
        pltpu.SemaphoreType.DMA,
        pltpu.SemaphoreType.DMA,
        pltpu.SemaphoreType.DMA,
        pltpu.SemaphoreType.DMA,
        pltpu.SemaphoreType.DMA,
    ],
)(_sc_body)


def _tcmat_body(nidx_ref, x_ref, weight_hbm, xw_ref, wn_v, sem):
    cps = []
    for k in range(_KPAD):
        idx = nidx_ref[min(k, _K - 1)]
        cps.append(pltpu.make_async_copy(
            weight_hbm.at[pl.ds(idx, 1), :], wn_v.at[pl.ds(k, 1), :], sem))
        cps[-1].start()
    for cp in cps:
        cp.wait()
    xw_ref[...] = lax.dot_general(
        x_ref[...], wn_v[...], (((1,), (1,)), ((), ())),
        preferred_element_type=jnp.float32)


def _tc_matmul(x2d, weight, noise_idx):
    return pl.pallas_call(
        _tcmat_body,
        in_specs=[
            pl.BlockSpec(memory_space=pltpu.SMEM),
            pl.BlockSpec(memory_space=pltpu.VMEM),
            pl.BlockSpec(memory_space=pl.ANY),
        ],
        out_shape=jax.ShapeDtypeStruct((_BN, _KPAD), jnp.float32),
        scratch_shapes=[
            pltpu.VMEM((_KPAD, _D), jnp.float32),
            pltpu.SemaphoreType.DMA,
        ],
    )(noise_idx, x2d, weight)


def _tcfin_body(xw_ref, ts_ref, ptn_ref, bn_ref, nn_ref, out_ref):
    ns = xw_ref[:, :_K] + jnp.reshape(bn_ref[...], (1, _K))
    pnim = jnp.exp(jnp.minimum(ns - _NORM, 20.0))
    kpn = jnp.reshape(_K * nn_ref[...], (1, _K))
    pnc = kpn / (pnim + kpn + _EPS)
    term2 = jnp.sum(jnp.log(pnc + _EPS))
    pm = jnp.exp(jnp.minimum(ts_ref[...] - _NORM, 20.0))
    pt = pm / (pm + _K * ptn_ref[...] + _EPS)
    term1 = jnp.sum(jnp.log(pt + _EPS))
    out_ref[...] = jnp.reshape(-(term1 + term2) / _BN, (1, 1))


def _tc_final(xw, ts, ptn, bn, nn):
    return pl.pallas_call(
        _tcfin_body,
        out_shape=jax.ShapeDtypeStruct((1, 1), jnp.float32),
    )(xw, ts, ptn, bn, nn)


def kernel(target, input, weight, bias, noise, noise_idx):
    x2d = input.reshape(_BN, _D)
    tflat = target.reshape(_BN).astype(jnp.int32)
    nidx = noise_idx.astype(jnp.int32)
    tscore, ptn, bn, nn = _sc_gather(weight, bias, noise, tflat, x2d, nidx)
    xw = _tc_matmul(x2d, weight, nidx)
    out = _tc_final(xw, tscore, ptn, bn, nn)
    return out[0, 0]


# speedup vs baseline: 1.8827x; 1.0135x over previous
import functools
import math

import jax
import jax.numpy as jnp
from jax import lax
from jax.experimental import pallas as pl
from jax.experimental.pallas import tpu as pltpu
from jax.experimental.pallas import tpu_sc as plsc

_V = 100000
_D = 1024
_K = 100
_KPAD = 128
_BN = 1024
_NORM = math.log(_V)
_EPS = 1e-10
_NC = 2
_NS = 16
_NW = _NC * _NS
_TPW = _BN // _NW
_L = 16
_NCH = 4
_CT = _TPW // _NCH
_JU = 8


def _sc_body(weight_hbm, bias_hbm, noise_hbm, target_hbm, x_hbm, nidx_hbm,
             tscore_hbm, ptn_hbm, bn_hbm, nn_hbm,
             idx_v, rows_v, x_v, bias_v, ptn_v, partial_v, score_v,
             kidx_v, bn_v, nn_v,
             sem_idx, sem_kidx, sem_rows, sem_x, sem_bias, sem_ptn,
             sem_bn, sem_nn, sem_po, sem_bno, sem_nno):
    cid = lax.axis_index("c")
    sid = lax.axis_index("s")
    wid = sid * _NC + cid
    base = wid * _TPW
    noise_worker = wid == _NW - 1

    idx_cp = pltpu.async_copy(target_hbm.at[wid], idx_v, sem_idx)

    @pl.when(noise_worker)
    def _():
        pltpu.async_copy(nidx_hbm, kidx_v, sem_kidx)

    idx_cp.wait()

    rows_cps = []
    x_cps = []
    for c in range(_NCH):
        rows_cps.append(pltpu.async_copy(
            weight_hbm.at[idx_v.at[pl.ds(c * _CT, _CT)]],
            rows_v.at[pl.ds(c * _CT, _CT), :], sem_rows[c]))
        x_cps.append(pltpu.async_copy(
            x_hbm.at[wid, pl.ds(c * _CT, _CT), :],
            x_v.at[pl.ds(c * _CT, _CT), :], sem_x[c]))
    bias_cp = pltpu.async_copy(bias_hbm.at[idx_v], bias_v, sem_bias)
    ptn_cp = pltpu.async_copy(noise_hbm.at[idx_v], ptn_v, sem_ptn)

    @pl.when(noise_worker)
    def _():
        pltpu.make_async_copy(nidx_hbm, kidx_v, sem_kidx).wait()
        pltpu.async_copy(bias_hbm.at[kidx_v], bn_v, sem_bn)
        pltpu.async_copy(noise_hbm.at[kidx_v], nn_v, sem_nn)

    for c in range(_NCH):
        rows_cps[c].wait()
        x_cps[c].wait()

        def tok_body(t, carry):
            def j_body(jo, acc):
                for u in range(_JU):
                    off = jo * (_JU * _L) + u * _L
                    acc = acc + (rows_v[t, pl.ds(off, _L)] *
                                 x_v[t, pl.ds(off, _L)])
                return acc

            acc = lax.fori_loop(0, _D // (_JU * _L), j_body,
                                jnp.zeros((_L,), jnp.float32))
            partial_v[t, :] = acc
            return carry

        lax.fori_loop(c * _CT, (c + 1) * _CT, tok_body, 0)

    @pl.when(noise_worker)
    def _():
        pltpu.make_async_copy(bias_hbm.at[kidx_v], bn_v, sem_bn).wait()
        pltpu.make_async_copy(noise_hbm.at[kidx_v], nn_v, sem_nn).wait()
        pltpu.async_copy(bn_v, bn_hbm, sem_bno)
        pltpu.async_copy(nn_v, nn_hbm, sem_nno)

    ptn_cp.wait()
    pltpu.async_copy(ptn_v, ptn_hbm.at[pl.ds(base, _TPW)], sem_po)
    bias_cp.wait()

    lanes = lax.iota(jnp.int32, _L)
    for g in range(_TPW // _L):
        row_ids = lanes + g * _L
        acc16 = jnp.zeros((_L,), jnp.float32)
        for j in range(_L):
            col_ids = jnp.full((_L,), j, jnp.int32)
            acc16 = acc16 + plsc.load_gather(partial_v, [row_ids, col_ids])
        score_v[pl.ds(g * _L, _L)] = acc16 + bias_v[pl.ds(g * _L, _L)]

    pltpu.sync_copy(score_v, tscore_hbm.at[pl.ds(base, _TPW)])
    pltpu.make_async_copy(ptn_v, ptn_hbm.at[pl.ds(base, _TPW)], sem_po).wait()

    @pl.when(noise_worker)
    def _():
        pltpu.make_async_copy(bn_v, bn_hbm, sem_bno).wait()
        pltpu.make_async_copy(nn_v, nn_hbm, sem_nno).wait()


_sc_gather = functools.partial(
    pl.kernel,
    out_type=[
        jax.ShapeDtypeStruct((_BN,), jnp.float32),
        jax.ShapeDtypeStruct((_BN,), jnp.float32),
        jax.ShapeDtypeStruct((_K,), jnp.float32),
        jax.ShapeDtypeStruct((_K,), jnp.float32),
    ],
    mesh=plsc.VectorSubcoreMesh(core_axis_name="c", subcore_axis_name="s"),
    compiler_params=pltpu.CompilerParams(needs_layout_passes=False),
    scratch_types=[
        pltpu.VMEM((_TPW,), jnp.int32),
        pltpu.VMEM((_TPW, _D), jnp.float32),
        pltpu.VMEM((_TPW, _D), jnp.float32),
        pltpu.VMEM((_TPW,), jnp.float32),
        pltpu.VMEM((_TPW,), jnp.float32),
        pltpu.VMEM((_TPW, _L), jnp.float32),
        pltpu.VMEM((_TPW,), jnp.float32),
        pltpu.VMEM((_K,), jnp.int32),
        pltpu.VMEM((_K,), jnp.float32),
        pltpu.VMEM((_K,), jnp.float32),
        pltpu.SemaphoreType.DMA,
        pltpu.SemaphoreType.DMA,
        [pltpu.SemaphoreType.DMA] * _NCH,
        [pltpu.SemaphoreType.DMA] * _NCH,
        pltpu.SemaphoreType.DMA,
        pltpu.SemaphoreType.DMA,
        pltpu.SemaphoreType.DMA,
        pltpu.SemaphoreType.DMA,
        pltpu.SemaphoreType.DMA,
        pltpu.SemaphoreType.DMA,
        pltpu.SemaphoreType.DMA,
    ],
)(_sc_body)


def _tcmat_body(nidx_ref, x_ref, weight_hbm, xw_ref, wn_v, sem):
    cps = []
    for k in range(_KPAD):
        idx = nidx_ref[min(k, _K - 1)]
        cps.append(pltpu.make_async_copy(
            weight_hbm.at[pl.ds(idx, 1), :], wn_v.at[pl.ds(k, 1), :], sem))
        cps[-1].start()
    for cp in cps:
        cp.wait()
    xs = jnp.reshape(x_ref[...], (_BN, _D))
    xw_ref[...] = lax.dot_general(
        xs, wn_v[...], (((1,), (1,)), ((), ())),
        preferred_element_type=jnp.float32)


def _tc_matmul(x3d, weight, noise_idx):
    return pl.pallas_call(
        _tcmat_body,
        in_specs=[
            pl.BlockSpec(memory_space=pltpu.SMEM),
            pl.BlockSpec(memory_space=pltpu.VMEM),
            pl.BlockSpec(memory_space=pl.ANY),
        ],
        out_shape=jax.ShapeDtypeStruct((_BN, _KPAD), jnp.float32),
        scratch_shapes=[
            pltpu.VMEM((_KPAD, _D), jnp.float32),
            pltpu.SemaphoreType.DMA,
        ],
    )(noise_idx, x3d, weight)


def _tcfin_body(xw_ref, ts_ref, ptn_ref, bn_ref, nn_ref, out_ref):
    ns = xw_ref[:, :_K] + jnp.reshape(bn_ref[...], (1, _K))
    pnim = jnp.exp(jnp.minimum(ns - _NORM, 20.0))
    kpn = jnp.reshape(_K * nn_ref[...], (1, _K))
    pnc = kpn / (pnim + kpn + _EPS)
    term2 = jnp.sum(jnp.log(pnc + _EPS))
    pm = jnp.exp(jnp.minimum(ts_ref[...] - _NORM, 20.0))
    pt = pm / (pm + _K * ptn_ref[...] + _EPS)
    term1 = jnp.sum(jnp.log(pt + _EPS))
    out_ref[...] = jnp.reshape(-(term1 + term2) / _BN, (1, 1))


def _tc_final(xw, ts, ptn, bn, nn):
    return pl.pallas_call(
        _tcfin_body,
        out_shape=jax.ShapeDtypeStruct((1, 1), jnp.float32),
    )(xw, ts, ptn, bn, nn)


def kernel(target, input, weight, bias, noise, noise_idx):
    nidx = noise_idx.astype(jnp.int32)
    tscore, ptn, bn, nn = _sc_gather(weight, bias, noise,
                                     target.astype(jnp.int32), input, nidx)
    xw = _tc_matmul(input, weight, nidx)
    out = _tc_final(xw, tscore, ptn, bn, nn)
    return out[0, 0]
